# baseline (speedup 1.0000x reference)
import functools

import jax
import jax.numpy as jnp
from jax import lax
from jax.experimental import pallas as pl
from jax.experimental.pallas import tpu as pltpu
from jax.experimental.pallas import tpu_sc as plsc

C_LEN = 1_000_000
U_LEN = 100_000
EMBED = 64
BATCH = 16384

NUM_CORES = 2
NUM_SUBCORES = 16
NW = NUM_CORES * NUM_SUBCORES
BPW = BATCH // NW
CHUNK = 128
NCH = BPW // CHUNK
LANES = 16
VPC = CHUNK // LANES

_mesh = plsc.VectorSubcoreMesh(core_axis_name="c", subcore_axis_name="s")


@functools.partial(
    pl.kernel,
    mesh=_mesh,
    out_type=jax.ShapeDtypeStruct((BATCH,), jnp.float32),
    compiler_params=pltpu.CompilerParams(
        needs_layout_passes=False, use_tc_tiling_on_sc=False),
    scratch_types=[
        pltpu.VMEM((BPW,), jnp.int32),
        pltpu.VMEM((BPW,), jnp.int32),
        pltpu.VMEM((EMBED, CHUNK), jnp.float32),
        pltpu.VMEM((EMBED, CHUNK), jnp.float32),
        pltpu.VMEM((BPW,), jnp.float32),
        pltpu.SemaphoreType.DMA,
    ],
)
def _gmf_sc(c_idx_hbm, u_idx_hbm, c_flat_hbm, u_flat_hbm, out_hbm,
            cidx_v, uidx_v, cdst, udst, out_v, sem):
    wid = lax.axis_index("s") * NUM_CORES + lax.axis_index("c")
    base = wid * BPW

    pltpu.sync_copy(c_idx_hbm.at[pl.ds(base, BPW)], cidx_v)
    pltpu.sync_copy(u_idx_hbm.at[pl.ds(base, BPW)], uidx_v)

    def chunk_body(ch, carry):
        cil = cidx_v.at[pl.ds(ch * CHUNK, CHUNK)]
        uil = uidx_v.at[pl.ds(ch * CHUNK, CHUNK)]
        copies = []
        for d in range(EMBED):
            copies.append(pltpu.async_copy(
                c_flat_hbm.at[pl.ds(d * C_LEN, C_LEN)].at[cil],
                cdst.at[d], sem))
            copies.append(pltpu.async_copy(
                u_flat_hbm.at[pl.ds(d * U_LEN, U_LEN)].at[uil],
                udst.at[d], sem))
        for cp in copies:
            cp.wait()

        for v in range(VPC):
            s = pl.ds(v * LANES, LANES)
            accs = [None] * 4
            for d in range(EMBED):
                p = cdst[d, s] * udst[d, s]
                k = d % 4
                accs[k] = p if accs[k] is None else accs[k] + p
            out_v[pl.ds(ch * CHUNK + v * LANES, LANES)] = (
                (accs[0] + accs[1]) + (accs[2] + accs[3]))
        return carry

    lax.fori_loop(0, NCH, chunk_body, 0)

    pltpu.sync_copy(out_v, out_hbm.at[pl.ds(base, BPW)])


def kernel(c_idx, u_idx, c_table, u_table):
    c_idx32 = jnp.asarray(c_idx, jnp.int32)
    u_idx32 = jnp.asarray(u_idx, jnp.int32)
    c_flat = c_table.T.reshape(C_LEN * EMBED)
    u_flat = u_table.T.reshape(U_LEN * EMBED)
    out = _gmf_sc(c_idx32, u_idx32, c_flat, u_flat)
    return out.reshape(BATCH, 1)


# speedup vs baseline: 1.0031x; 1.0031x over previous
import functools

import jax
import jax.numpy as jnp
from jax import lax
from jax.experimental import pallas as pl
from jax.experimental.pallas import tpu as pltpu
from jax.experimental.pallas import tpu_sc as plsc

C_LEN = 1_000_000
U_LEN = 100_000
EMBED = 64
BATCH = 16384

NUM_CORES = 2
NUM_SUBCORES = 16
NW = NUM_CORES * NUM_SUBCORES
BPW = BATCH // NW
CHUNK = 128
NCH = BPW // CHUNK
LANES = 16
VPC = CHUNK // LANES

_mesh = plsc.VectorSubcoreMesh(core_axis_name="c", subcore_axis_name="s")


@functools.partial(
    pl.kernel,
    mesh=_mesh,
    out_type=jax.ShapeDtypeStruct((BATCH,), jnp.float32),
    compiler_params=pltpu.CompilerParams(
        needs_layout_passes=False, use_tc_tiling_on_sc=False),
    scratch_types=[
        pltpu.VMEM((BPW,), jnp.int32),
        pltpu.VMEM((BPW,), jnp.int32),
        pltpu.VMEM((EMBED, CHUNK), jnp.float32),
        pltpu.VMEM((EMBED, CHUNK), jnp.float32),
        pltpu.VMEM((BPW,), jnp.float32),
        pltpu.SemaphoreType.DMA,
    ],
)
def _gmf_sc(c_idx_hbm, u_idx_hbm, c_tab_hbm, u_tab_hbm, out_hbm,
            cidx_v, uidx_v, cdst, udst, out_v, sem):
    wid = lax.axis_index("s") * NUM_CORES + lax.axis_index("c")
    base = wid * BPW

    pltpu.sync_copy(c_idx_hbm.at[pl.ds(base, BPW)], cidx_v)
    pltpu.sync_copy(u_idx_hbm.at[pl.ds(base, BPW)], uidx_v)

    def chunk_body(ch, carry):
        cil = cidx_v.at[pl.ds(ch * CHUNK, CHUNK)]
        uil = uidx_v.at[pl.ds(ch * CHUNK, CHUNK)]
        copies = []
        for d in range(EMBED):
            copies.append(pltpu.async_copy(
                c_tab_hbm.at[d].at[cil], cdst.at[d], sem))
            copies.append(pltpu.async_copy(
                u_tab_hbm.at[d].at[uil], udst.at[d], sem))
        for cp in copies:
            cp.wait()

        for v in range(VPC):
            s = pl.ds(v * LANES, LANES)
            accs = [None] * 4
            for d in range(EMBED):
                p = cdst[d, s] * udst[d, s]
                k = d % 4
                accs[k] = p if accs[k] is None else accs[k] + p
            out_v[pl.ds(ch * CHUNK + v * LANES, LANES)] = (
                (accs[0] + accs[1]) + (accs[2] + accs[3]))
        return carry

    lax.fori_loop(0, NCH, chunk_body, 0)

    pltpu.sync_copy(out_v, out_hbm.at[pl.ds(base, BPW)])


def kernel(c_idx, u_idx, c_table, u_table):
    c_idx32 = jnp.asarray(c_idx, jnp.int32)
    u_idx32 = jnp.asarray(u_idx, jnp.int32)
    out = _gmf_sc(c_idx32, u_idx32, c_table.T, u_table.T)
    return out.reshape(BATCH, 1)
